# baseline (speedup 1.0000x reference)
import jax
import jax.numpy as jnp
import numpy as np
from jax.experimental import pallas as pl
from jax.experimental.pallas import tpu as pltpu


def _pick_bm(n):
    for bm in (400, 200, 100, 50, 8):
        if n % bm == 0:
            return bm
    return n


def _prop_block(adj_ref, v_ref, o_ref):
    o_ref[...] = jnp.dot(adj_ref[...], v_ref[...],
                         preferred_element_type=jnp.float32)


def _propagate(adj, v):
    n = adj.shape[0]
    d = v.shape[1]
    bm = _pick_bm(n)
    return pl.pallas_call(
        _prop_block,
        grid=(n // bm,),
        in_specs=[
            pl.BlockSpec((bm, n), lambda i: (i, 0)),
            pl.BlockSpec((n, d), lambda i: (0, 0)),
        ],
        out_specs=pl.BlockSpec((bm, d), lambda i: (i, 0)),
        out_shape=jax.ShapeDtypeStruct((n, d), jnp.float32),
        compiler_params=pltpu.CompilerParams(
            dimension_semantics=("parallel",),
        ),
    )(adj, v)


def _make_tri_body(n, d, bm, bk, ct, co):
    def _body(tbl_ref, adja_ref, adjb_ref, x_ref, wt_ref, b_ref, o_ref,
              t_ref, ot_ref):
        s = pl.program_id(0)
        phase = tbl_ref[s, 0]
        al = tbl_ref[s, 1]
        bl = tbl_ref[s, 2]
        c = tbl_ref[s, 3]
        dual = tbl_ref[s, 4]
        flag = tbl_ref[s, 5]
        lo = tbl_ref[s, 6]
        hi = tbl_ref[s, 7]
        ol = tbl_ref[s, 8]

        @pl.when(s == 0)
        def _init():
            t_ref[...] = jnp.zeros(t_ref.shape, jnp.float32)
            ot_ref[...] = jnp.zeros(ot_ref.shape, jnp.float32)

        @pl.when((phase == 0) & (dual == 1))
        def _a_dual():
            ot_ref[pl.ds(al * bm, bm), :] += jnp.dot(
                adja_ref[...], t_ref[pl.ds(0, n), :],
                preferred_element_type=jnp.float32)

        @pl.when(phase == 0)
        def _a_t():
            t_ref[pl.ds(al * bm, bm), :] = jnp.dot(
                adja_ref[...], x_ref[...],
                preferred_element_type=jnp.float32)

        @pl.when((phase == 1) & (flag == 0))
        def _b_plain():
            ot_ref[pl.ds(bl * bm, bm), :] += jnp.dot(
                adjb_ref[...], t_ref[pl.ds(c * bk, bk), :],
                preferred_element_type=jnp.float32)

        @pl.when((phase == 1) & (flag == 1))
        def _b_masked():
            col = jax.lax.broadcasted_iota(jnp.int32, (bm, bk), 1)
            a = jnp.where((col >= lo) & (col < hi), adjb_ref[...], 0.0)
            ot_ref[pl.ds(bl * bm, bm), :] += jnp.dot(
                a, t_ref[pl.ds(c * bk, bk), :],
                preferred_element_type=jnp.float32)

        @pl.when(phase == 2)
        def _c():
            o_ref[...] = jnp.dot(
                ot_ref[pl.ds(ol * co, co), :], wt_ref[...],
                preferred_element_type=jnp.float32) + b_ref[...]

    return _body


def _propagate2_linear_tri(adj, v, wt, b2):
    n = adj.shape[0]
    d = v.shape[1]
    d_out = wt.shape[1]
    bm = _pick_bm(n)
    nb = n // bm
    bk = min(2048, 128 * max(1, n // 128))
    ct = -(-n // bk)
    co = 1000 if n % 1000 == 0 else bm
    nc = n // co

    rows = []
    for l in range(nb):
        rows.append((0, l, 0, 0, 1 if l > 0 else 0, 0, 0, 0, 0))
    for l in range(nb):
        c0 = (bm * l) // bk
        for c in range(c0, ct):
            lo = max(0, bm * l - bk * c)
            hi = min(bk, n - bk * c)
            flag = 1 if (lo > 0 or hi < bk) else 0
            rows.append((1, nb - 1, l, c, 0, flag, lo, hi, 0))
    last = rows[-1]
    for ol in range(nc):
        rows.append((2, nb - 1, last[2], last[3], 0, 0, 0, 0, ol))
    tbl = jnp.asarray(np.asarray(rows, dtype=np.int32))

    grid_spec = pltpu.PrefetchScalarGridSpec(
        num_scalar_prefetch=1,
        grid=(len(rows),),
        in_specs=[
            pl.BlockSpec((bm, n), lambda s, tbl: (tbl[s, 1], 0)),
            pl.BlockSpec((bm, bk), lambda s, tbl: (tbl[s, 2], tbl[s, 3])),
            pl.BlockSpec((n, d), lambda s, tbl: (0, 0)),
            pl.BlockSpec((d, d_out), lambda s, tbl: (0, 0)),
            pl.BlockSpec((1, d_out), lambda s, tbl: (0, 0)),
        ],
        out_specs=pl.BlockSpec((co, d_out), lambda s, tbl: (tbl[s, 8], 0)),
        scratch_shapes=[
            pltpu.VMEM((ct * bk, d), jnp.float32),
            pltpu.VMEM((n, d), jnp.float32),
        ],
    )
    return pl.pallas_call(
        _make_tri_body(n, d, bm, bk, ct, co),
        grid_spec=grid_spec,
        out_shape=jax.ShapeDtypeStruct((n, d_out), jnp.float32),
        compiler_params=pltpu.CompilerParams(
            dimension_semantics=("arbitrary",),
        ),
    )(tbl, adj, adj, v, wt, b2)


def kernel(x, adj, W, b, k):
    h = jax.lax.fori_loop(0, k - 2, lambda i, h: _propagate(adj, h), x)
    return _propagate2_linear_tri(adj, h, W.T, b.reshape(1, -1))


# speedup vs baseline: 1.0413x; 1.0413x over previous
import jax
import jax.numpy as jnp
import numpy as np
from jax.experimental import pallas as pl
from jax.experimental.pallas import tpu as pltpu


def _pick_bm(n):
    for bm in (400, 200, 100, 50, 8):
        if n % bm == 0:
            return bm
    return n


def _prop_block(adj_ref, v_ref, o_ref):
    o_ref[...] = jnp.dot(adj_ref[...], v_ref[...],
                         preferred_element_type=jnp.float32)


def _propagate(adj, v):
    n = adj.shape[0]
    d = v.shape[1]
    bm = _pick_bm(n)
    return pl.pallas_call(
        _prop_block,
        grid=(n // bm,),
        in_specs=[
            pl.BlockSpec((bm, n), lambda i: (i, 0)),
            pl.BlockSpec((n, d), lambda i: (0, 0)),
        ],
        out_specs=pl.BlockSpec((bm, d), lambda i: (i, 0)),
        out_shape=jax.ShapeDtypeStruct((n, d), jnp.float32),
        compiler_params=pltpu.CompilerParams(
            dimension_semantics=("parallel",),
        ),
    )(adj, v)


def _make_tri_body(n, d, bm, bk, ct, co):
    def _body(tbl_ref, adja_ref, adjb_ref, x_ref, wt_ref, b_ref, o_ref,
              xt_ref, ot_ref):
        s = pl.program_id(0)
        phase = tbl_ref[s, 0]
        al = tbl_ref[s, 1]
        bl = tbl_ref[s, 2]
        c = tbl_ref[s, 3]
        flag = tbl_ref[s, 5]
        lo = tbl_ref[s, 6]
        hi = tbl_ref[s, 7]
        ol = tbl_ref[s, 8]

        @pl.when(s == 0)
        def _init():
            xt_ref[...] = jnp.zeros(xt_ref.shape, jnp.float32)
            xt_ref[pl.ds(0, n), pl.ds(0, d)] = x_ref[...]
            ot_ref[...] = jnp.zeros(ot_ref.shape, jnp.float32)

        @pl.when(phase == 0)
        def _a():
            r = jnp.dot(adja_ref[...], xt_ref[pl.ds(0, n), :],
                        preferred_element_type=jnp.float32)
            ot_ref[pl.ds(al * bm, bm), :] += r[:, d:]
            xt_ref[pl.ds(al * bm, bm), pl.ds(d, d)] = r[:, :d]

        @pl.when((phase == 1) & (flag == 0))
        def _b_plain():
            ot_ref[pl.ds(bl * bm, bm), :] += jnp.dot(
                adjb_ref[...], xt_ref[pl.ds(c * bk, bk), pl.ds(d, d)],
                preferred_element_type=jnp.float32)

        @pl.when((phase == 1) & (flag == 1))
        def _b_masked():
            col = jax.lax.broadcasted_iota(jnp.int32, (bm, bk), 1)
            a = jnp.where((col >= lo) & (col < hi), adjb_ref[...], 0.0)
            ot_ref[pl.ds(bl * bm, bm), :] += jnp.dot(
                a, xt_ref[pl.ds(c * bk, bk), pl.ds(d, d)],
                preferred_element_type=jnp.float32)

        @pl.when(phase == 2)
        def _c():
            o_ref[...] = jnp.dot(
                ot_ref[pl.ds(ol * co, co), :], wt_ref[...],
                preferred_element_type=jnp.float32) + b_ref[...]

    return _body


def _propagate2_linear_tri(adj, v, wt, b2):
    n = adj.shape[0]
    d = v.shape[1]
    d_out = wt.shape[1]
    bm = _pick_bm(n)
    nb = n // bm
    bk = min(2048, 128 * max(1, n // 128))
    ct = -(-n // bk)
    co = 1000 if n % 1000 == 0 else bm
    nc = n // co

    rows = []
    for l in range(nb):
        rows.append((0, l, 0, 0, 1 if l > 0 else 0, 0, 0, 0, 0))
    for l in range(nb):
        c0 = (bm * l) // bk
        for c in range(c0, ct):
            lo = max(0, bm * l - bk * c)
            hi = min(bk, n - bk * c)
            flag = 1 if (lo > 0 or hi < bk) else 0
            rows.append((1, nb - 1, l, c, 0, flag, lo, hi, 0))
    last = rows[-1]
    for ol in range(nc):
        rows.append((2, nb - 1, last[2], last[3], 0, 0, 0, 0, ol))
    tbl = jnp.asarray(np.asarray(rows, dtype=np.int32))

    grid_spec = pltpu.PrefetchScalarGridSpec(
        num_scalar_prefetch=1,
        grid=(len(rows),),
        in_specs=[
            pl.BlockSpec((bm, n), lambda s, tbl: (tbl[s, 1], 0)),
            pl.BlockSpec((bm, bk), lambda s, tbl: (tbl[s, 2], tbl[s, 3])),
            pl.BlockSpec((n, d), lambda s, tbl: (0, 0)),
            pl.BlockSpec((d, d_out), lambda s, tbl: (0, 0)),
            pl.BlockSpec((1, d_out), lambda s, tbl: (0, 0)),
        ],
        out_specs=pl.BlockSpec((co, d_out), lambda s, tbl: (tbl[s, 8], 0)),
        scratch_shapes=[
            pltpu.VMEM((ct * bk, 2 * d), jnp.float32),
            pltpu.VMEM((n, d), jnp.float32),
        ],
    )
    return pl.pallas_call(
        _make_tri_body(n, d, bm, bk, ct, co),
        grid_spec=grid_spec,
        out_shape=jax.ShapeDtypeStruct((n, d_out), jnp.float32),
        compiler_params=pltpu.CompilerParams(
            dimension_semantics=("arbitrary",),
        ),
    )(tbl, adj, adj, v, wt, b2)


def kernel(x, adj, W, b, k):
    h = jax.lax.fori_loop(0, k - 2, lambda i, h: _propagate(adj, h), x)
    return _propagate2_linear_tri(adj, h, W.T, b.reshape(1, -1))


# speedup vs baseline: 1.0805x; 1.0376x over previous
import jax
import jax.numpy as jnp
import numpy as np
from jax.experimental import pallas as pl
from jax.experimental.pallas import tpu as pltpu


def _pick_bm(n):
    for bm in (400, 200, 100, 50, 8):
        if n % bm == 0:
            return bm
    return n


def _prop_block(adj_ref, v_ref, o_ref):
    o_ref[...] = jnp.dot(adj_ref[...], v_ref[...],
                         preferred_element_type=jnp.float32)


def _propagate(adj, v):
    n = adj.shape[0]
    d = v.shape[1]
    bm = _pick_bm(n)
    return pl.pallas_call(
        _prop_block,
        grid=(n // bm,),
        in_specs=[
            pl.BlockSpec((bm, n), lambda i: (i, 0)),
            pl.BlockSpec((n, d), lambda i: (0, 0)),
        ],
        out_specs=pl.BlockSpec((bm, d), lambda i: (i, 0)),
        out_shape=jax.ShapeDtypeStruct((n, d), jnp.float32),
        compiler_params=pltpu.CompilerParams(
            dimension_semantics=("parallel",),
        ),
    )(adj, v)


def _make_tri_body(n, d, bm, bk, ct, co):
    def _body(tbl_ref, adja_ref, adjb_ref, x_ref, wt_ref, b_ref, o_ref,
              xt_ref, ot_ref):
        s = pl.program_id(0)
        phase = tbl_ref[s, 0]
        al = tbl_ref[s, 1]
        bl = tbl_ref[s, 2]
        c = tbl_ref[s, 3]
        fl = tbl_ref[s, 4]
        flag = tbl_ref[s, 5]
        lo = tbl_ref[s, 6]
        hi = tbl_ref[s, 7]
        ol = tbl_ref[s, 8]

        @pl.when(s == 0)
        def _init():
            xt_ref[...] = jnp.zeros(xt_ref.shape, jnp.float32)
            xt_ref[pl.ds(0, n), pl.ds(0, d)] = x_ref[...]
            ot_ref[...] = jnp.zeros(ot_ref.shape, jnp.float32)

        @pl.when(phase == 0)
        def _a():
            r = jnp.dot(adja_ref[...], xt_ref[pl.ds(0, n), :],
                        preferred_element_type=jnp.float32)
            ot_ref[pl.ds(al * bm, bm), :] += r[:, d:]
            xt_ref[pl.ds(al * bm, bm), pl.ds(d, d)] = r[:, :d]

        @pl.when((phase == 1) & (flag == 0))
        def _b_plain():
            ot_ref[pl.ds(bl * bm, bm), :] += jnp.dot(
                adjb_ref[...], xt_ref[pl.ds(c * bk, bk), pl.ds(d, d)],
                preferred_element_type=jnp.float32)

        @pl.when((phase == 1) & (flag == 1))
        def _b_masked():
            col = jax.lax.broadcasted_iota(jnp.int32, (bm, bk), 1)
            a = jnp.where((col >= lo) & (col < hi), adjb_ref[...], 0.0)
            ot_ref[pl.ds(bl * bm, bm), :] += jnp.dot(
                a, xt_ref[pl.ds(c * bk, bk), pl.ds(d, d)],
                preferred_element_type=jnp.float32)

        @pl.when((phase == 3) & (flag == 1))
        def _bf_zero():
            xt_ref[pl.ds((fl - 1) * bm, bm), pl.ds(d, d)] = jnp.zeros(
                (bm, d), jnp.float32)

        @pl.when(phase == 3)
        def _bf():
            ot_ref[pl.ds(fl * bm, bm), :] += jnp.dot(
                adja_ref[...], xt_ref[pl.ds(0, n), pl.ds(d, d)],
                preferred_element_type=jnp.float32)

        @pl.when(phase == 2)
        def _c():
            o_ref[...] = jnp.dot(
                ot_ref[pl.ds(ol * co, co), :], wt_ref[...],
                preferred_element_type=jnp.float32) + b_ref[...]

    return _body


def _propagate2_linear_tri(adj, v, wt, b2):
    n = adj.shape[0]
    d = v.shape[1]
    d_out = wt.shape[1]
    bm = _pick_bm(n)
    nb = n // bm
    bk = min(2048, 128 * max(1, n // 128))
    ct = -(-n // bk)
    co = 1000 if n % 1000 == 0 else bm
    nc = n // co

    rows = []
    for l in range(nb):
        rows.append((0, l, 0, 0, 0, 0, 0, 0, 0))
    bfull = [l for l in range(nb) if (bm * l) // bk == 0]
    btile = [l for l in range(nb) if (bm * l) // bk >= 1]
    for l in btile:
        c0 = (bm * l) // bk
        for c in range(c0, ct):
            lo = max(0, bm * l - bk * c)
            hi = min(bk, n - bk * c)
            flag = 1 if (lo > 0 or hi < bk) else 0
            rows.append((1, nb - 1, l, c, 0, flag, lo, hi, 0))
    lastb = rows[-1]
    for i, l in enumerate(bfull):
        rows.append((3, l, lastb[2], lastb[3], l, 1 if i > 0 else 0, 0, 0, 0))
    lasta = rows[-1]
    for ol in range(nc):
        rows.append((2, lasta[1], lastb[2], lastb[3], 0, 0, 0, 0, ol))
    tbl = jnp.asarray(np.asarray(rows, dtype=np.int32))

    grid_spec = pltpu.PrefetchScalarGridSpec(
        num_scalar_prefetch=1,
        grid=(len(rows),),
        in_specs=[
            pl.BlockSpec((bm, n), lambda s, tbl: (tbl[s, 1], 0)),
            pl.BlockSpec((bm, bk), lambda s, tbl: (tbl[s, 2], tbl[s, 3])),
            pl.BlockSpec((n, d), lambda s, tbl: (0, 0)),
            pl.BlockSpec((d, d_out), lambda s, tbl: (0, 0)),
            pl.BlockSpec((1, d_out), lambda s, tbl: (0, 0)),
        ],
        out_specs=pl.BlockSpec((co, d_out), lambda s, tbl: (tbl[s, 8], 0)),
        scratch_shapes=[
            pltpu.VMEM((ct * bk, 2 * d), jnp.float32),
            pltpu.VMEM((n, d), jnp.float32),
        ],
    )
    return pl.pallas_call(
        _make_tri_body(n, d, bm, bk, ct, co),
        grid_spec=grid_spec,
        out_shape=jax.ShapeDtypeStruct((n, d_out), jnp.float32),
        compiler_params=pltpu.CompilerParams(
            dimension_semantics=("arbitrary",),
        ),
    )(tbl, adj, adj, v, wt, b2)


def kernel(x, adj, W, b, k):
    h = jax.lax.fori_loop(0, k - 2, lambda i, h: _propagate(adj, h), x)
    return _propagate2_linear_tri(adj, h, W.T, b.reshape(1, -1))


# speedup vs baseline: 1.0999x; 1.0180x over previous
import jax
import jax.numpy as jnp
import numpy as np
from jax.experimental import pallas as pl
from jax.experimental.pallas import tpu as pltpu


def _pick_bm(n):
    for bm in (400, 200, 100, 50, 8):
        if n % bm == 0:
            return bm
    return n


def _prop_block(adj_ref, v_ref, o_ref):
    o_ref[...] = jnp.dot(adj_ref[...], v_ref[...],
                         preferred_element_type=jnp.float32)


def _propagate(adj, v):
    n = adj.shape[0]
    d = v.shape[1]
    bm = _pick_bm(n)
    return pl.pallas_call(
        _prop_block,
        grid=(n // bm,),
        in_specs=[
            pl.BlockSpec((bm, n), lambda i: (i, 0)),
            pl.BlockSpec((n, d), lambda i: (0, 0)),
        ],
        out_specs=pl.BlockSpec((bm, d), lambda i: (i, 0)),
        out_shape=jax.ShapeDtypeStruct((n, d), jnp.float32),
        compiler_params=pltpu.CompilerParams(
            dimension_semantics=("parallel",),
        ),
    )(adj, v)


def _make_tri_body(n, d, bm, bk, ct, co):
    def _body(tbl_ref, adja_ref, adjb_ref, x_ref, wt_ref, b_ref, o_ref,
              xt_ref, ot_ref):
        s = pl.program_id(0)
        phase = tbl_ref[s, 0]
        al = tbl_ref[s, 1]
        bl = tbl_ref[s, 2]
        c = tbl_ref[s, 3]
        fl = tbl_ref[s, 4]
        flag = tbl_ref[s, 5]
        lo = tbl_ref[s, 6]
        hi = tbl_ref[s, 7]
        ol = tbl_ref[s, 8]

        @pl.when(s == 0)
        def _init():
            xt_ref[...] = jnp.zeros(xt_ref.shape, jnp.float32)
            xt_ref[pl.ds(0, n), pl.ds(0, d)] = x_ref[...]
            ot_ref[...] = jnp.zeros(ot_ref.shape, jnp.float32)

        @pl.when(phase == 0)
        def _a():
            r = jnp.dot(adja_ref[...], xt_ref[pl.ds(0, n), :],
                        preferred_element_type=jnp.float32)
            ot_ref[pl.ds(al * bm, bm), :] += r[:, d:]
            xt_ref[pl.ds(al * bm, bm), pl.ds(d, d)] = r[:, :d]

        @pl.when((phase == 1) & (flag == 0))
        def _b_plain():
            ot_ref[pl.ds(bl * bm, bm), :] += jnp.dot(
                adjb_ref[...], xt_ref[pl.ds(c * bk, bk), pl.ds(d, d)],
                preferred_element_type=jnp.float32)

        @pl.when((phase == 1) & (flag == 1))
        def _b_masked():
            col = jax.lax.broadcasted_iota(jnp.int32, (bm, bk), 1)
            a = jnp.where((col >= lo) & (col < hi), adjb_ref[...], 0.0)
            ot_ref[pl.ds(bl * bm, bm), :] += jnp.dot(
                a, xt_ref[pl.ds(c * bk, bk), pl.ds(d, d)],
                preferred_element_type=jnp.float32)

        @pl.when((phase == 3) & (flag == 1))
        def _bf_zero():
            xt_ref[pl.ds((fl - 1) * bm, bm), pl.ds(d, d)] = jnp.zeros(
                (bm, d), jnp.float32)

        @pl.when(phase == 3)
        def _bf():
            ot_ref[pl.ds(fl * bm, bm), :] += jnp.dot(
                adja_ref[...], xt_ref[pl.ds(0, n), pl.ds(d, d)],
                preferred_element_type=jnp.float32)

        @pl.when(phase == 2)
        def _c():
            o_ref[...] = jnp.dot(
                ot_ref[pl.ds(ol * co, co), :], wt_ref[...],
                preferred_element_type=jnp.float32) + b_ref[...]

    return _body


def _propagate2_linear_tri(adj, v, wt, b2):
    n = adj.shape[0]
    d = v.shape[1]
    d_out = wt.shape[1]
    bm = _pick_bm(n)
    nb = n // bm
    bk = min(2048, 128 * max(1, n // 128))
    ct = -(-n // bk)
    co = 1000 if n % 1000 == 0 else bm
    nc = n // co

    rows = []
    for l in range(nb):
        rows.append((0, l, 0, 0, 0, 0, 0, 0, 0))
    bfull = [l for l in range(nb) if (bm * l) // bk <= 1]
    btile = [l for l in range(nb) if (bm * l) // bk >= 2]
    for l in btile:
        c0 = (bm * l) // bk
        for c in range(c0, ct):
            lo = max(0, bm * l - bk * c)
            hi = min(bk, n - bk * c)
            flag = 1 if (lo > 0 or hi < bk) else 0
            rows.append((1, nb - 1, l, c, 0, flag, lo, hi, 0))
    lastb = rows[-1]
    for i, l in enumerate(bfull):
        rows.append((3, l, lastb[2], lastb[3], l, 1 if i > 0 else 0, 0, 0, 0))
    lasta = rows[-1]
    for ol in range(nc):
        rows.append((2, lasta[1], lastb[2], lastb[3], 0, 0, 0, 0, ol))
    tbl = jnp.asarray(np.asarray(rows, dtype=np.int32))

    grid_spec = pltpu.PrefetchScalarGridSpec(
        num_scalar_prefetch=1,
        grid=(len(rows),),
        in_specs=[
            pl.BlockSpec((bm, n), lambda s, tbl: (tbl[s, 1], 0)),
            pl.BlockSpec((bm, bk), lambda s, tbl: (tbl[s, 2], tbl[s, 3])),
            pl.BlockSpec((n, d), lambda s, tbl: (0, 0)),
            pl.BlockSpec((d, d_out), lambda s, tbl: (0, 0)),
            pl.BlockSpec((1, d_out), lambda s, tbl: (0, 0)),
        ],
        out_specs=pl.BlockSpec((co, d_out), lambda s, tbl: (tbl[s, 8], 0)),
        scratch_shapes=[
            pltpu.VMEM((ct * bk, 2 * d), jnp.float32),
            pltpu.VMEM((n, d), jnp.float32),
        ],
    )
    return pl.pallas_call(
        _make_tri_body(n, d, bm, bk, ct, co),
        grid_spec=grid_spec,
        out_shape=jax.ShapeDtypeStruct((n, d_out), jnp.float32),
        compiler_params=pltpu.CompilerParams(
            dimension_semantics=("arbitrary",),
        ),
    )(tbl, adj, adj, v, wt, b2)


def kernel(x, adj, W, b, k):
    h = jax.lax.fori_loop(0, k - 2, lambda i, h: _propagate(adj, h), x)
    return _propagate2_linear_tri(adj, h, W.T, b.reshape(1, -1))
